# baseline (speedup 1.0000x reference)
import functools

import jax
import jax.numpy as jnp
from jax import lax
from jax.experimental import pallas as pl
from jax.experimental.pallas import tpu as pltpu
from jax.experimental.pallas import tpu_sc as plsc

VOCAB = 1000000
EMBED = 64
MAXLEN = 200
BATCH = 4096
B = BATCH * MAXLEN
NW = 32
BPW = B // NW
CHUNK = MAXLEN
NCH = BPW // CHUNK
NLANES = 16

_mesh = plsc.VectorSubcoreMesh(core_axis_name="c", subcore_axis_name="s")


@functools.partial(
    pl.kernel,
    out_type=jax.ShapeDtypeStruct((B, EMBED), jnp.float32),
    mesh=_mesh,
    scratch_types=[
        pltpu.VMEM((MAXLEN, EMBED), jnp.float32),
        pltpu.VMEM((CHUNK,), jnp.int32),
        pltpu.VMEM((CHUNK, EMBED), jnp.float32),
        pltpu.SemaphoreType.DMA,
    ],
    compiler_params=pltpu.CompilerParams(use_tc_tiling_on_sc=False),
)
def _embed_sc(x_hbm, table_hbm, fix_hbm, out_hbm, fix_v, idx_v, rows_v, sem):
    wid = lax.axis_index("s") * 2 + lax.axis_index("c")
    pltpu.sync_copy(fix_hbm, fix_v)

    def chunk_body(c, _):
        base = wid * BPW + c * CHUNK
        pltpu.sync_copy(x_hbm.at[pl.ds(base, CHUNK)], idx_v)
        pltpu.async_copy(table_hbm.at[idx_v], rows_v, sem).wait()

        def row_body(i, _):
            for j in range(EMBED // NLANES):
                sl = pl.ds(j * NLANES, NLANES)
                rows_v[i, sl] = rows_v[i, sl] + fix_v[i, sl]
            return ()

        lax.fori_loop(0, CHUNK, row_body, ())
        pltpu.sync_copy(rows_v, out_hbm.at[pl.ds(base, CHUNK)])
        return ()

    lax.fori_loop(0, NCH, chunk_body, ())


def kernel(x, input_table, fix_embedding):
    out = _embed_sc(x.reshape(B), input_table, fix_embedding)
    return out.reshape(BATCH, MAXLEN, EMBED)


# speedup vs baseline: 1.2092x; 1.2092x over previous
import functools

import jax
import jax.numpy as jnp
from jax import lax
from jax.experimental import pallas as pl
from jax.experimental.pallas import tpu as pltpu
from jax.experimental.pallas import tpu_sc as plsc

VOCAB = 1000000
EMBED = 64
MAXLEN = 200
BATCH = 4096
B = BATCH * MAXLEN
NW = 32
BPW = B // NW
CHUNK = 2 * MAXLEN
NCH = BPW // CHUNK
NBUF = 4
NLANES = 16
RPI = 4

_mesh = plsc.VectorSubcoreMesh(core_axis_name="c", subcore_axis_name="s")


@functools.partial(
    pl.kernel,
    out_type=jax.ShapeDtypeStruct((B, EMBED), jnp.float32),
    mesh=_mesh,
    scratch_types=[
        pltpu.VMEM((MAXLEN, EMBED), jnp.float32),
        [pltpu.VMEM((CHUNK,), jnp.int32) for _ in range(NBUF)],
        [pltpu.VMEM((CHUNK, EMBED), jnp.float32) for _ in range(NBUF)],
        [pltpu.SemaphoreType.DMA for _ in range(NBUF)],
        [pltpu.SemaphoreType.DMA for _ in range(NBUF)],
        [pltpu.SemaphoreType.DMA for _ in range(NBUF)],
    ],
    compiler_params=pltpu.CompilerParams(use_tc_tiling_on_sc=False),
)
def _embed_sc(x_hbm, table_hbm, fix_hbm, out_hbm,
              fix_v, idx_v, rows_v, isem, gsem, ssem):
    wid = lax.axis_index("s") * 2 + lax.axis_index("c")
    wbase = wid * BPW
    pltpu.sync_copy(fix_hbm, fix_v)

    def idx_start(b, c):
        pltpu.async_copy(x_hbm.at[pl.ds(wbase + c * CHUNK, CHUNK)],
                         idx_v[b], isem[b])

    def idx_wait(b, c):
        pltpu.make_async_copy(x_hbm.at[pl.ds(wbase + c * CHUNK, CHUNK)],
                              idx_v[b], isem[b]).wait()

    def gather_start(b):
        pltpu.async_copy(table_hbm.at[idx_v[b]], rows_v[b], gsem[b])

    def gather_wait(b):
        pltpu.make_async_copy(table_hbm.at[idx_v[b]], rows_v[b],
                              gsem[b]).wait()

    def store_start(b, c):
        pltpu.async_copy(rows_v[b],
                         out_hbm.at[pl.ds(wbase + c * CHUNK, CHUNK)], ssem[b])

    def store_wait(b, c):
        pltpu.make_async_copy(rows_v[b],
                              out_hbm.at[pl.ds(wbase + c * CHUNK, CHUNK)],
                              ssem[b]).wait()

    def add_fix(b):
        rows = rows_v[b]
        for rep in range(CHUNK // MAXLEN):
            def body(i, _):
                for r in range(RPI):
                    fr = i * RPI + r
                    for j in range(EMBED // NLANES):
                        sl = pl.ds(j * NLANES, NLANES)
                        rows[rep * MAXLEN + fr, sl] = (
                            rows[rep * MAXLEN + fr, sl] + fix_v[fr, sl])
                return ()
            lax.fori_loop(0, MAXLEN // RPI, body, ())

    for b in range(3):
        idx_start(b, b)
    for b in range(2):
        idx_wait(b, b)
        gather_start(b)

    def group_body(g, _):
        for b in range(NBUF):
            c = g * NBUF + b
            bi = (b + 3) % NBUF

            @pl.when(c + 3 < NCH)
            def _():
                idx_start(bi, c + 3)

            gather_wait(b)
            add_fix(b)
            store_start(b, c)

            bg = (b + 2) % NBUF

            @pl.when(c + 2 < NCH)
            def _():
                @pl.when(c >= 2)
                def _():
                    store_wait(bg, c - 2)
                idx_wait(bg, c + 2)
                gather_start(bg)
        return ()

    lax.fori_loop(0, NCH // NBUF, group_body, ())

    for b in range(NBUF):
        store_wait(b, NCH - NBUF + b)


def kernel(x, input_table, fix_embedding):
    out = _embed_sc(x.reshape(B), input_table, fix_embedding)
    return out.reshape(BATCH, MAXLEN, EMBED)
